# Initial kernel scaffold; baseline (speedup 1.0000x reference)
import jax
import jax.numpy as jnp
from jax.experimental import pallas as pl


def kernel(x, edge_index, batch, global_features, W0, b0, W1l, W1r, att1, bias1, W2l, W2r, att2, bias2, W3l, W3r, att3, bias3, W4l, W4r, att4, bias4, g1, be1, g2, be2, g3, be3, g4, be4, fcW1, fcb1, fcW2, fcb2, fcW3, fcb3):
    raise NotImplementedError("write your pallas kernel here")


# speedup vs baseline: 35.8935x; 35.8935x over previous
import functools

import jax
import jax.numpy as jnp
from jax import lax
from jax.experimental import pallas as pl
from jax.experimental.pallas import tpu as pltpu
from jax.experimental.pallas import tpu_sc as plsc

H = 8
C = 16
F = H * C
N = 10000
G = 8

NCORE = 2
NSUB = 16
NW = NCORE * NSUB

NP = 10240
RPW = NP // NSUB

E0 = 160000
ET = E0 + N
CH = 64
NCHUNK = -(-ET // (NW * CH))
EPW = NCHUNK * CH
EP = NW * EPW

RB = 512
NBLK = NP // RB

@functools.lru_cache(maxsize=1)
def _build_sc_edge_pass():
  mesh = plsc.VectorSubcoreMesh(core_axis_name="c", subcore_axis_name="s",
                                num_cores=NCORE, num_subcores=NSUB)

  @functools.partial(
    pl.kernel,
    out_type=[jax.ShapeDtypeStruct((NCORE, NP, F), jnp.float32),
              jax.ShapeDtypeStruct((NCORE, NP * H), jnp.float32)],
    mesh=mesh,
    scratch_types=[
        pltpu.VMEM((CH,), jnp.int32),
        pltpu.VMEM((CH,), jnp.int32),
        pltpu.VMEM((CH + 16,), jnp.int32),
        pltpu.VMEM((CH, F), jnp.float32),
        pltpu.VMEM((CH, F), jnp.float32),
        pltpu.VMEM((CH, F), jnp.float32),
        pltpu.VMEM((H, C), jnp.float32),
        pltpu.VMEM((CH * H // 128, 128), jnp.float32),
        pltpu.VMEM((CH * H // 128, 128), jnp.int32),
        pltpu.VMEM_SHARED((NP, F), jnp.float32),
        pltpu.VMEM_SHARED((NP * H,), jnp.float32),
        pltpu.SemaphoreType.DMA,
        pltpu.SemaphoreType.DMA,
    ],
    compiler_params=pltpu.CompilerParams(needs_layout_passes=False),
  )
  def _sc_body(xl, xr, src, dst, att, zz, zzd, out, out_den,
               sidx, didx, didxp, xlg, xrg, msgs, att_v, dstage, istage,
               acc, den, sem1, sem2):
    cid = lax.axis_index("c")
    sid = lax.axis_index("s")
    wid = cid * NSUB + sid
    dpw = NP * H // NSUB

    pltpu.sync_copy(att, att_v)
    pltpu.sync_copy(zzd.at[pl.ds(sid * dpw, dpw)], den.at[pl.ds(sid * dpw, dpw)])
    pltpu.sync_copy(zz.at[pl.ds(sid * RPW, RPW)], acc.at[pl.ds(sid * RPW, RPW)])
    plsc.subcore_barrier()

    iot = lax.iota(jnp.int32, 16)
    lane_is = [iot == h for h in range(H)]
    att_rows = [att_v[h, :] for h in range(H)]
    perms = [iot ^ sft for sft in (1, 2, 4, 8)]

    def lane_total(t):
        for p in perms:
            t = t + t.at[p].get(mode="promise_in_bounds")
        return t

    ebase = wid * EPW

    @pl.loop(0, NCHUNK)
    def _chunk(ci):
        off = ebase + ci * CH
        pltpu.sync_copy(src.at[pl.ds(off, CH)], sidx)
        pltpu.sync_copy(dst.at[pl.ds(off, CH)], didx)
        pltpu.sync_copy(dst.at[pl.ds(off, CH)], didxp.at[pl.ds(0, CH)])
        cp1 = pltpu.async_copy(xl.at[sidx], xlg, sem1)
        cp2 = pltpu.async_copy(xr.at[didx], xrg, sem2)
        cp1.wait()
        cp2.wait()

        @plsc.parallel_loop(0, CH, unroll=2)
        def _edge(e):
            tail = jnp.zeros((16,), jnp.float32)
            for h in range(H):
                xlv = xlg[e, pl.ds(C * h, C)]
                xrv = xrg[e, pl.ds(C * h, C)]
                a = xlv + xrv
                lr = jnp.maximum(a, 0.2 * a)
                t = lr * att_rows[h]
                eh = jnp.exp(lane_total(t))
                msgs[e, pl.ds(C * h, C)] = xlv * eh
                tail = jnp.where(lane_is[h], eh, tail)
            dv = didxp[pl.ds(e, 16)]
            fidx = jnp.broadcast_to(dv[0], (16,)) * H + iot
            srow = jnp.broadcast_to(e // 16, (16,))
            scol = jnp.broadcast_to((e % 16) * H, (16,)) + iot
            lmask = iot < H
            plsc.store_scatter(dstage, [srow, scol], tail, mask=lmask)
            plsc.store_scatter(istage, [srow, scol], fidx, mask=lmask)

        pltpu.sync_copy(msgs, acc.at[didx], add=True)
        for j in range(CH * H // 128):
            pltpu.sync_copy(dstage.at[j], den.at[istage.at[j]], add=True)

    plsc.subcore_barrier()
    pltpu.sync_copy(acc.at[pl.ds(sid * RPW, RPW)],
                    out.at[cid, pl.ds(sid * RPW, RPW)])
    pltpu.sync_copy(den.at[pl.ds(sid * dpw, dpw)],
                    out_den.at[cid, pl.ds(sid * dpw, dpw)])

  return _sc_body


def _sc_edge_call(xl, xr, srcp, dstp, att, zz, zzd):
    return _build_sc_edge_pass()(xl, xr, srcp, dstp, att, zz, zzd)


def _embed_body(xp_ref, w0_ref, b0_ref, wl_ref, wr_ref, xl_ref, xr_ref):
    i = pl.program_id(0)
    h0 = jnp.maximum(
        jnp.dot(xp_ref[...], w0_ref[...], preferred_element_type=jnp.float32)
        + b0_ref[...], 0.0)
    rows = i * RB + lax.broadcasted_iota(jnp.int32, (RB, 1), 0)
    h0 = jnp.where(rows < N, h0, 0.0)
    xl_ref[...] = jnp.dot(h0, wl_ref[...], preferred_element_type=jnp.float32)
    xr_ref[...] = jnp.dot(h0, wr_ref[...], preferred_element_type=jnp.float32)


def _embed_call(xp, w0p, b0r, wl, wr):
    return pl.pallas_call(
        _embed_body,
        grid=(NBLK,),
        in_specs=[
            pl.BlockSpec((RB, 8), lambda i: (i, 0)),
            pl.BlockSpec((8, 64), lambda i: (0, 0)),
            pl.BlockSpec((1, 64), lambda i: (0, 0)),
            pl.BlockSpec((64, F), lambda i: (0, 0)),
            pl.BlockSpec((64, F), lambda i: (0, 0)),
        ],
        out_specs=[
            pl.BlockSpec((RB, F), lambda i: (i, 0)),
            pl.BlockSpec((RB, F), lambda i: (i, 0)),
        ],
        out_shape=[
            jax.ShapeDtypeStruct((NP, F), jnp.float32),
            jax.ShapeDtypeStruct((NP, F), jnp.float32),
        ],
    )(xp, w0p, b0r, wl, wr)


def _gat_block(acc_blk, den_blk, bias_ref, i):
    msg = acc_blk[0] + acc_blk[1]
    den = jnp.sum(den_blk, axis=0)
    dfull = jnp.broadcast_to(den[:, :, None], (RB, H, C)).reshape(RB, F)
    rows = i * RB + lax.broadcasted_iota(jnp.int32, (RB, 1), 0)
    return jnp.where(rows < N, msg / dfull + bias_ref[...], 0.0)


def _post_body(res, last, acc_ref, den_ref, hprev_ref, bias_ref, g_ref, be_ref,
               wl_ref, wr_ref, h_ref, xl_ref, xr_ref, stats):
    p = pl.program_id(0)
    i = pl.program_id(1)
    gat = _gat_block(acc_ref[...], den_ref[...], bias_ref, i)

    @pl.when(p == 0)
    def _accum():
        @pl.when(i == 0)
        def _init():
            stats[...] = jnp.zeros_like(stats)
        stats[0:1, :] += jnp.sum(gat, axis=0, keepdims=True)
        stats[1:2, :] += jnp.sum(gat * gat, axis=0, keepdims=True)

    @pl.when(p == 1)
    def _apply():
        mean = stats[0:1, :] * (1.0 / N)
        var = stats[1:2, :] * (1.0 / N) - mean * mean
        scale = g_ref[...] / jnp.sqrt(var + 1e-5)
        y = (gat - mean) * scale + be_ref[...]
        if res:
            y = y + hprev_ref[...]
        rows = i * RB + lax.broadcasted_iota(jnp.int32, (RB, 1), 0)
        hh = jnp.where(rows < N, jnp.maximum(y, 0.0), 0.0)
        h_ref[...] = hh
        if not last:
            xl_ref[...] = jnp.dot(hh, wl_ref[...],
                                  preferred_element_type=jnp.float32)
            xr_ref[...] = jnp.dot(hh, wr_ref[...],
                                  preferred_element_type=jnp.float32)


def _post_call(acc, den, hprev, biasr, gr, ber, wl, wr, res, last):
    body = functools.partial(_post_body, res, last)
    full = lambda p, i: (0, 0)
    out_specs = [pl.BlockSpec((RB, F), lambda p, i: (i, 0))] * 3
    out_shape = [jax.ShapeDtypeStruct((NP, F), jnp.float32)] * 3
    outs = pl.pallas_call(
        body,
        grid=(2, NBLK),
        in_specs=[
            pl.BlockSpec((NCORE, RB, F), lambda p, i: (0, i, 0)),
            pl.BlockSpec((NCORE, RB, H), lambda p, i: (0, i, 0)),
            pl.BlockSpec((RB, F), lambda p, i: (i, 0)),
            pl.BlockSpec((1, F), full),
            pl.BlockSpec((1, F), full),
            pl.BlockSpec((1, F), full),
            pl.BlockSpec((F, F), full),
            pl.BlockSpec((F, F), full),
        ],
        out_specs=out_specs,
        out_shape=out_shape,
        scratch_shapes=[pltpu.VMEM((8, F), jnp.float32)],
    )(acc, den, hprev, biasr, gr, ber, wl, wr)
    return outs


def _head_body(h_ref, bo_ref, gf_ref, w1a_ref, w1b_ref, b1_ref,
               w2_ref, b2_ref, w3_ref, b3_ref, out_ref, pool, cnt):
    i = pl.program_id(0)

    @pl.when(i == 0)
    def _init():
        pool[...] = jnp.zeros_like(pool)
        cnt[...] = jnp.zeros_like(cnt)

    hb = h_ref[...]
    bob = bo_ref[...]
    dn = (((0,), (0,)), ((), ()))
    pool[...] += lax.dot_general(bob, hb, dn,
                                 preferred_element_type=jnp.float32)
    cnt[...] += lax.dot_general(bob, jnp.ones_like(hb), dn,
                                preferred_element_type=jnp.float32)

    @pl.when(i == NBLK - 1)
    def _finish():
        pooled = pool[...] / jnp.maximum(cnt[...], 1.0)
        z = jnp.dot(pooled, w1a_ref[...], preferred_element_type=jnp.float32)
        z += jnp.dot(gf_ref[...], w1b_ref[...],
                     preferred_element_type=jnp.float32)
        z = jnp.maximum(z + b1_ref[...], 0.0)
        z = jnp.maximum(
            jnp.dot(z, w2_ref[...], preferred_element_type=jnp.float32)
            + b2_ref[...], 0.0)
        out_ref[...] = jnp.dot(z, w3_ref[...],
                               preferred_element_type=jnp.float32) + b3_ref[...]


def _head_call(h, bo, gf, w1a, w1b, b1r, w2, b2r, w3p, b3r):
    full = lambda i: (0, 0)
    return pl.pallas_call(
        _head_body,
        grid=(NBLK,),
        in_specs=[
            pl.BlockSpec((RB, F), lambda i: (i, 0)),
            pl.BlockSpec((RB, G), lambda i: (i, 0)),
            pl.BlockSpec((G, 16), full),
            pl.BlockSpec((F, F), full),
            pl.BlockSpec((16, F), full),
            pl.BlockSpec((1, F), full),
            pl.BlockSpec((F, 64), full),
            pl.BlockSpec((1, 64), full),
            pl.BlockSpec((64, F), full),
            pl.BlockSpec((1, F), full),
        ],
        out_specs=pl.BlockSpec((G, F), full),
        out_shape=jax.ShapeDtypeStruct((G, F), jnp.float32),
        scratch_shapes=[pltpu.VMEM((G, F), jnp.float32),
                        pltpu.VMEM((G, F), jnp.float32)],
    )(h, bo, gf, w1a, w1b, b1r, w2, b2r, w3p, b3r)


def kernel(x, edge_index, batch, global_features, W0, b0,
           W1l, W1r, att1, bias1, W2l, W2r, att2, bias2,
           W3l, W3r, att3, bias3, W4l, W4r, att4, bias4,
           g1, be1, g2, be2, g3, be3, g4, be4,
           fcW1, fcb1, fcW2, fcb2, fcW3, fcb3):
    f32 = jnp.float32
    i32 = jnp.int32

    xp = jnp.pad(x, ((0, NP - N), (0, 8 - x.shape[1])))
    w0p = jnp.pad(W0, ((0, 8 - W0.shape[0]), (0, 0)))

    loop = jnp.arange(N, dtype=i32)
    npad = EP - ET
    srcp = jnp.concatenate(
        [edge_index[0].astype(i32), loop, jnp.zeros((npad,), i32)])
    dstp = jnp.concatenate(
        [edge_index[1].astype(i32), loop, jnp.full((npad,), N, i32)])

    zz = jnp.zeros((NP, F), f32)
    zzd = jnp.zeros((NP * H,), f32)
    row2 = lambda v: v.reshape(1, -1)

    xl, xr = _embed_call(xp, w0p, row2(b0), W1l, W1r)

    layers = [
        (att1, bias1, g1, be1, W2l, W2r, False, False),
        (att2, bias2, g2, be2, W3l, W3r, True, False),
        (att3, bias3, g3, be3, W4l, W4r, True, False),
        (att4, bias4, g4, be4, W4l, W4r, True, True),
    ]
    h = jnp.zeros((NP, F), f32)
    for att, bias, g, be, wln, wrn, res, last in layers:
        acc, den = _sc_edge_call(xl, xr, srcp, dstp, att, zz, zzd)
        den = den.reshape(NCORE, NP, H)
        h, xl, xr = _post_call(acc, den, h, row2(bias), row2(g), row2(be),
                               wln, wrn, res, last)

    batch_pad = jnp.concatenate([batch.astype(i32), jnp.full((NP - N,), G, i32)])
    bo = (batch_pad[:, None] == jnp.arange(G, dtype=i32)[None, :]).astype(f32)

    w3p = jnp.pad(fcW3, ((0, 0), (0, F - fcW3.shape[1])))
    b3r = jnp.pad(fcb3, (0, F - fcb3.shape[0])).reshape(1, F)
    out = _head_call(h, bo, global_features, fcW1[:F], fcW1[F:], row2(fcb1),
                     fcW2, row2(fcb2), w3p, b3r)
    return out[:, :1]
